# baseline (speedup 1.0000x reference)
import functools

import jax
import jax.numpy as jnp
import numpy as np
from jax import lax
from jax.experimental import pallas as pl
from jax.experimental.pallas import tpu as pltpu
from jax.experimental.pallas import tpu_sc as plsc

_FIELD_DIMS = [100000] * 26
_F = 26
_D = 16
_T = 4
_B = 4096
_BF = _B * _F

_info = plsc.get_sparse_core_info()
_NC, _NS = _info.num_cores, _info.num_subcores
_NW = _NC * _NS
_NPW = _BF // _NW


@functools.partial(
    pl.kernel,
    out_type=[
        jax.ShapeDtypeStruct((_BF, _D), jnp.float32),
        jax.ShapeDtypeStruct((_BF,), jnp.float32),
    ],
    mesh=plsc.VectorSubcoreMesh(core_axis_name="c", subcore_axis_name="s"),
    compiler_params=pltpu.CompilerParams(use_tc_tiling_on_sc=False),
    scratch_types=[
        pltpu.VMEM((_NPW,), jnp.int32),
        pltpu.VMEM((_NPW, _D), jnp.float32),
        pltpu.VMEM((_NPW,), jnp.float32),
        pltpu.SemaphoreType.DMA,
        pltpu.SemaphoreType.DMA,
    ],
)
def _sc_gather(idx_hbm, emb2_hbm, emb1_hbm, e_out, l_out, idx_v, rows_v, v1_v, s2, s1):
    wid = lax.axis_index("s") * _NC + lax.axis_index("c")
    base = wid * _NPW
    pltpu.sync_copy(idx_hbm.at[pl.ds(base, _NPW)], idx_v)
    c2 = pltpu.async_copy(emb2_hbm.at[idx_v], rows_v, s2)
    c1 = pltpu.async_copy(emb1_hbm.at[idx_v], v1_v, s1)
    c2.wait()
    pltpu.sync_copy(rows_v, e_out.at[pl.ds(base, _NPW)])
    c1.wait()
    pltpu.sync_copy(v1_v, l_out.at[pl.ds(base, _NPW)])


_BT = 128
_PAIRS = _F * (_F - 1) // 2


def _tc_body(eT_ref, g1T_ref, const_ref, out_ref):
    eT = eT_ref[...]
    C = const_ref[...]

    prods = []
    for i in range(_F - 1):
        cnt = _F - 1 - i
        left = eT[_D * i:_D * (i + 1), :]
        right = eT[_D * (i + 1):, :]
        lrep = jnp.concatenate([left] * cnt, axis=0)
        prods.append(lrep * right)
    P3 = jnp.concatenate(prods, axis=0).reshape(_PAIRS, _D, _BT)

    us = []
    for t in range(_T + 1):
        wt = C[_D * t:_D * (t + 1), :]
        us.append(jnp.sum(P3 * wt[None, :, :], axis=1))

    score = jnp.zeros((_PAIRS, _BT), jnp.float32)
    for t in range(_T):
        b1_t = C[80 + t:81 + t, :]
        w2_t = C[84 + t:85 + t, :]
        score = score + w2_t * jnp.maximum(us[t] + b1_t, 0.0)

    m = jnp.max(score, axis=0, keepdims=True)
    ex = jnp.exp(score - m)
    z = jnp.sum(ex, axis=0, keepdims=True)
    numer = jnp.sum(ex * us[_T], axis=0, keepdims=True)
    attr_part = numer / z

    lin = jnp.sum(g1T_ref[...], axis=0, keepdims=True)
    w0v = C[88:89, :]
    logit = w0v + lin + attr_part
    out = 1.0 / (1.0 + jnp.exp(-logit))
    out_ref[...] = jnp.broadcast_to(out, (8, _BT))


def _tc_compute(eT, g1T, const):
    grid = _B // _BT
    return pl.pallas_call(
        _tc_body,
        grid=(grid,),
        in_specs=[
            pl.BlockSpec((_F * _D, _BT), lambda i: (0, i)),
            pl.BlockSpec((_F, _BT), lambda i: (0, i)),
            pl.BlockSpec((96, _BT), lambda i: (0, 0)),
        ],
        out_specs=pl.BlockSpec((8, _BT), lambda i: (0, i)),
        out_shape=jax.ShapeDtypeStruct((8, _B), jnp.float32),
    )(eT, g1T, const)


def kernel(x, emb1, emb2, w0, p, W1, b1, W2):
    offsets = jnp.asarray(np.cumsum([0] + _FIELD_DIMS[:-1]), dtype=x.dtype)
    idx = (x + offsets[None, :]).reshape(-1)

    e_flat, lin_flat = _sc_gather(idx, emb2, emb1.reshape(-1))

    eT = e_flat.reshape(_B, _F * _D).T
    g1T = lin_flat.reshape(_B, _F).T

    W5 = jnp.concatenate([W1, p[:, None]], axis=1)
    top = jnp.repeat(W5.T.reshape(5 * _D, 1), _BT, axis=1)
    sc9 = jnp.concatenate([b1, W2[:, 0], w0, jnp.zeros((7,), jnp.float32)])
    bot = jnp.repeat(sc9.reshape(16, 1), _BT, axis=1)
    const = jnp.concatenate([top, bot], axis=0)

    o8 = _tc_compute(eT, g1T, const)
    return o8[0].reshape(_B, 1)
